# baseline (speedup 1.0000x reference)
import functools

import jax
import jax.numpy as jnp
from jax import lax
from jax.experimental import pallas as pl
from jax.experimental.pallas import tpu as pltpu
from jax.experimental.pallas import tpu_sc as plsc

B, S, D, P, E = 4, 2048, 1024, 256, 8
N = B * S
T = 1024
NT = N // T
G = NT + E - 1

NC, NS = 2, 16
NW = NC * NS
RPW = N // NW
XCH = 32
NXCH = RPW // XCH
OCH = 128
NOCH = RPW // OCH

@functools.cache
def _sc_kernels():
    mesh = plsc.VectorSubcoreMesh(core_axis_name="c", subcore_axis_name="s")

    @functools.partial(
        pl.kernel,
        mesh=mesh,
        out_type=jax.ShapeDtypeStruct((N, D), jnp.float32),
        scratch_types=[
            pltpu.VMEM((NXCH, XCH), jnp.int32),
            pltpu.VMEM((XCH, D), jnp.float32),
            pltpu.VMEM((XCH, D), jnp.float32),
            pltpu.SemaphoreType.DMA,
            pltpu.SemaphoreType.DMA,
            pltpu.SemaphoreType.DMA,
            pltpu.SemaphoreType.DMA,
        ],
    )
    def scatter_x(x_hbm, pos_hbm, xs_hbm, idx_v, rows0, rows1,
                  rs0, rs1, ws0, ws1):
        wid = lax.axis_index("s") * NC + lax.axis_index("c")
        base = wid * RPW
        pltpu.sync_copy(pos_hbm.at[wid], idx_v)
        bufs = (rows0, rows1)
        rsem = (rs0, rs1)
        wsem = (ws0, ws1)

        def read(j, b):
            return pltpu.async_copy(
                x_hbm.at[pl.ds(base + j * XCH, XCH)], bufs[b], rsem[b])

        h_r = [read(0, 0), read(1, 1)]
        h_w = [None, None]
        for j in range(NXCH):
            b = j % 2
            h_r[b].wait()
            h_w[b] = pltpu.async_copy(bufs[b], xs_hbm.at[idx_v.at[j]], wsem[b])
            if j + 2 < NXCH:
                h_w[b].wait()
                h_r[b] = read(j + 2, b)
        h_w[(NXCH - 2) % 2].wait()
        h_w[(NXCH - 1) % 2].wait()

    @functools.partial(
        pl.kernel,
        mesh=mesh,
        out_type=jax.ShapeDtypeStruct((N, P), jnp.float32),
        scratch_types=[
            pltpu.VMEM((NOCH, OCH), jnp.int32),
            pltpu.VMEM((OCH, P), jnp.float32),
            pltpu.VMEM((OCH, P), jnp.float32),
            pltpu.SemaphoreType.DMA,
            pltpu.SemaphoreType.DMA,
            pltpu.SemaphoreType.DMA,
            pltpu.SemaphoreType.DMA,
        ],
    )
    def gather_out(ys_hbm, pos_hbm, out_hbm, idx_v, rows0, rows1,
                   rs0, rs1, ws0, ws1):
        wid = lax.axis_index("s") * NC + lax.axis_index("c")
        base = wid * RPW
        pltpu.sync_copy(pos_hbm.at[wid], idx_v)
        bufs = (rows0, rows1)
        rsem = (rs0, rs1)
        wsem = (ws0, ws1)
        h_r = [pltpu.async_copy(ys_hbm.at[idx_v.at[j]], bufs[j], rsem[j])
               for j in range(NOCH)]
        h_w = []
        for j in range(NOCH):
            h_r[j].wait()
            h_w.append(pltpu.async_copy(
                bufs[j], out_hbm.at[pl.ds(base + j * OCH, OCH)], wsem[j]))
        for h in h_w:
            h.wait()

    return scatter_x, gather_out


def _mm_body(ti, ei, lo, hi, fi, xs_ref, w_ref, b_ref, out_ref):
    g = pl.program_id(0)
    e = ei[g]
    row0 = ti[g] * T
    rows = row0 + lax.broadcasted_iota(jnp.int32, (T, 1), 0)
    mask = (rows >= lo[g]) & (rows < hi[g])
    xb = xs_ref[...].astype(jnp.bfloat16)
    w = w_ref[e]
    y = lax.dot_general(
        xb, w,
        dimension_numbers=(((1,), (0,)), ((), ())),
        preferred_element_type=jnp.float32,
    )
    contrib = jnp.where(mask, y + b_ref[e], 0.0)

    @pl.when(fi[g] == 1)
    def _init():
        out_ref[...] = contrib

    @pl.when(fi[g] == 0)
    def _acc():
        out_ref[...] = out_ref[...] + contrib


_mm_grid_spec = pltpu.PrefetchScalarGridSpec(
    num_scalar_prefetch=5,
    grid=(G,),
    in_specs=[
        pl.BlockSpec((T, D), lambda g, ti, ei, lo, hi, fi: (ti[g], 0)),
        pl.BlockSpec((E, D, P), lambda g, ti, ei, lo, hi, fi: (0, 0, 0)),
        pl.BlockSpec((E, 1, P), lambda g, ti, ei, lo, hi, fi: (0, 0, 0)),
    ],
    out_specs=pl.BlockSpec((T, P), lambda g, ti, ei, lo, hi, fi: (ti[g], 0)),
)

_grouped_matmul = pl.pallas_call(
    _mm_body,
    grid_spec=_mm_grid_spec,
    out_shape=jax.ShapeDtypeStruct((N, P), jnp.float32),
)


def _routing_metadata(idx_flat):
    eids = jnp.arange(E, dtype=jnp.int32)
    onehot = (idx_flat[:, None] == eids[None, :]).astype(jnp.int32)
    cum = jnp.cumsum(onehot, axis=0)
    counts = cum[-1]
    starts = jnp.concatenate(
        [jnp.zeros((1,), jnp.int32), jnp.cumsum(counts)[:-1].astype(jnp.int32)])
    ends = starts + counts
    rank = jnp.sum(onehot * cum, axis=1) - 1
    pos = jnp.sum(onehot * starts[None, :], axis=1) + rank

    valid = counts > 0
    ft = jnp.where(valid, starts // T, 0)
    lt = jnp.where(valid, (ends - 1) // T, 0)
    n_e = jnp.where(valid, lt - ft + 1, 0)
    step_start = jnp.concatenate(
        [jnp.zeros((1,), jnp.int32), jnp.cumsum(n_e)[:-1].astype(jnp.int32)])
    nsteps = jnp.sum(n_e)

    g = jnp.arange(G, dtype=jnp.int32)
    in_e = ((g[:, None] >= step_start[None, :])
            & (g[:, None] < (step_start + n_e)[None, :]))
    in_e_i = in_e.astype(jnp.int32)
    e_of_g = jnp.sum(in_e_i * eids[None, :], axis=1)
    start_g = jnp.sum(in_e_i * starts[None, :], axis=1)
    end_g = jnp.sum(in_e_i * ends[None, :], axis=1)
    j = g - jnp.sum(in_e_i * step_start[None, :], axis=1)
    tile = jnp.sum(in_e_i * ft[None, :], axis=1) + j

    is_real = g < nsteps
    tile_ids = jnp.where(is_real, tile, NT - 1).astype(jnp.int32)
    expert_ids = jnp.where(is_real, e_of_g, E - 1).astype(jnp.int32)
    lo = jnp.where(is_real, jnp.maximum(start_g, tile_ids * T), 0).astype(jnp.int32)
    hi = jnp.where(is_real, jnp.minimum(end_g, (tile_ids + 1) * T), 0).astype(jnp.int32)
    first = (is_real & (lo == tile_ids * T)).astype(jnp.int32)
    return pos.astype(jnp.int32), tile_ids, expert_ids, lo, hi, first


def kernel(x, block_indices, W, b):
    xf = x.reshape(N, D)
    idx_flat = block_indices.reshape(N).astype(jnp.int32)
    pos, tile_ids, expert_ids, lo, hi, first = _routing_metadata(idx_flat)
    scatter_x, gather_out = _sc_kernels()
    xs = scatter_x(xf, pos.reshape(NW, NXCH, XCH))
    ys = _grouped_matmul(tile_ids, expert_ids, lo, hi, first,
                         xs, W.astype(jnp.bfloat16).transpose(0, 2, 1),
                         b.reshape(E, 1, P))
    out = gather_out(ys, pos.reshape(NW, NOCH, OCH))
    return out.reshape(B, S, P)


# speedup vs baseline: 1.0790x; 1.0790x over previous
import functools

import jax
import jax.numpy as jnp
from jax import lax
from jax.experimental import pallas as pl
from jax.experimental.pallas import tpu as pltpu
from jax.experimental.pallas import tpu_sc as plsc

B, S, D, P, E = 4, 2048, 1024, 256, 8
N = B * S
T = 512
NT = N // T
G = NT + E - 1

NC, NS = 2, 16
NW = NC * NS
RPW = N // NW
XCH = 32
NXCH = RPW // XCH
OCH = 128
NOCH = RPW // OCH

@functools.cache
def _sc_kernels():
    mesh = plsc.VectorSubcoreMesh(core_axis_name="c", subcore_axis_name="s")

    @functools.partial(
        pl.kernel,
        mesh=mesh,
        compiler_params=pltpu.CompilerParams(needs_layout_passes=False),
        out_type=(
            jax.ShapeDtypeStruct((N, D), jnp.float32),
            jax.ShapeDtypeStruct((NW, NXCH, XCH), jnp.int32),
        ),
        scratch_types=[
            pltpu.VMEM((RPW,), jnp.int32),
            pltpu.VMEM((E, 16), jnp.int32),
            pltpu.VMEM((NXCH, XCH), jnp.int32),
            pltpu.VMEM((XCH, D), jnp.float32),
            pltpu.VMEM((XCH, D), jnp.float32),
            pltpu.SemaphoreType.DMA,
            pltpu.SemaphoreType.DMA,
            pltpu.SemaphoreType.DMA,
            pltpu.SemaphoreType.DMA,
            pltpu.SemaphoreType.DMA,
        ],
    )
    def scatter_x(x_hbm, eid_hbm, base_hbm, xs_hbm, pos_hbm,
                  ids_v, state_v, idx_v, rows0, rows1,
                  rs0, rs1, ws0, ws1, psem):
        wid = lax.axis_index("s") * NC + lax.axis_index("c")
        base = wid * RPW
        bufs = (rows0, rows1)
        rsem = (rs0, rs1)
        wsem = (ws0, ws1)

        def read(j, b):
            return pltpu.async_copy(
                x_hbm.at[pl.ds(base + j * XCH, XCH)], bufs[b], rsem[b])

        h_r = [read(0, 0), read(1, 1)]
        pltpu.sync_copy(eid_hbm.at[wid], ids_v)
        pltpu.sync_copy(base_hbm.at[wid], state_v)

        zero = jnp.zeros((16,), jnp.int32)
        state = [state_v[e, :] for e in range(E)]
        for k in range(RPW // 16):
            ids = ids_v[pl.ds(k * 16, 16)]
            pos16 = zero
            for e in range(E):
                m = ids == e
                c = plsc.cumsum(jnp.where(m, 1, 0))
                pos16 = pos16 + jnp.where(m, state[e] + c - 1, 0)
                state[e] = state[e] + plsc.all_reduce_population_count(m)
            idx_v[k // 2, pl.ds((k % 2) * 16, 16)] = pos16

        h_p = pltpu.async_copy(idx_v, pos_hbm.at[wid], psem)
        h_w = [None, None]
        for j in range(NXCH):
            b = j % 2
            h_r[b].wait()
            h_w[b] = pltpu.async_copy(bufs[b], xs_hbm.at[idx_v.at[j]], wsem[b])
            if j + 2 < NXCH:
                h_w[b].wait()
                h_r[b] = read(j + 2, b)
        h_w[(NXCH - 2) % 2].wait()
        h_w[(NXCH - 1) % 2].wait()
        h_p.wait()

    @functools.partial(
        pl.kernel,
        mesh=mesh,
        out_type=jax.ShapeDtypeStruct((N, P), jnp.float32),
        scratch_types=[
            pltpu.VMEM((NOCH, OCH), jnp.int32),
            pltpu.VMEM((OCH, P), jnp.float32),
            pltpu.VMEM((OCH, P), jnp.float32),
            pltpu.SemaphoreType.DMA,
            pltpu.SemaphoreType.DMA,
            pltpu.SemaphoreType.DMA,
            pltpu.SemaphoreType.DMA,
        ],
    )
    def gather_out(ys_hbm, pos_hbm, out_hbm, idx_v, rows0, rows1,
                   rs0, rs1, ws0, ws1):
        wid = lax.axis_index("s") * NC + lax.axis_index("c")
        base = wid * RPW
        pltpu.sync_copy(pos_hbm.at[wid], idx_v)
        bufs = (rows0, rows1)
        rsem = (rs0, rs1)
        wsem = (ws0, ws1)
        h_r = [pltpu.async_copy(ys_hbm.at[idx_v.at[j]], bufs[j], rsem[j])
               for j in range(NOCH)]
        h_w = []
        for j in range(NOCH):
            h_r[j].wait()
            h_w.append(pltpu.async_copy(
                bufs[j], out_hbm.at[pl.ds(base + j * OCH, OCH)], wsem[j]))
        for h in h_w:
            h.wait()

    return scatter_x, gather_out


def _mm_body(ti, ei, lo, hi, fi, xs_ref, w_ref, b_ref, out_ref):
    g = pl.program_id(0)
    e = ei[g]
    row0 = ti[g] * T
    rows = row0 + lax.broadcasted_iota(jnp.int32, (T, 1), 0)
    mask = (rows >= lo[g]) & (rows < hi[g])
    xb = xs_ref[...].astype(jnp.bfloat16)
    w = w_ref[e]
    y = lax.dot_general(
        xb, w,
        dimension_numbers=(((1,), (0,)), ((), ())),
        preferred_element_type=jnp.float32,
    )
    contrib = jnp.where(mask, y + b_ref[e], 0.0)

    @pl.when(fi[g] == 1)
    def _init():
        out_ref[...] = contrib

    @pl.when(fi[g] == 0)
    def _acc():
        out_ref[...] = out_ref[...] + contrib


_mm_grid_spec = pltpu.PrefetchScalarGridSpec(
    num_scalar_prefetch=5,
    grid=(G,),
    in_specs=[
        pl.BlockSpec((T, D), lambda g, ti, ei, lo, hi, fi: (ti[g], 0)),
        pl.BlockSpec((E, D, P), lambda g, ti, ei, lo, hi, fi: (0, 0, 0)),
        pl.BlockSpec((E, 1, P), lambda g, ti, ei, lo, hi, fi: (0, 0, 0)),
    ],
    out_specs=pl.BlockSpec((T, P), lambda g, ti, ei, lo, hi, fi: (ti[g], 0)),
)

_grouped_matmul = pl.pallas_call(
    _mm_body,
    grid_spec=_mm_grid_spec,
    out_shape=jax.ShapeDtypeStruct((N, P), jnp.float32),
)


def _routing_metadata(idx_w):
    eids = jnp.arange(E, dtype=jnp.int32)
    onehot = (idx_w[:, :, None] == eids[None, None, :]).astype(jnp.int32)
    counts_wc = onehot.sum(1)
    counts = counts_wc.sum(0)
    starts = jnp.concatenate(
        [jnp.zeros((1,), jnp.int32), jnp.cumsum(counts)[:-1].astype(jnp.int32)])
    ends = starts + counts
    wbase = jnp.cumsum(counts_wc, axis=0) - counts_wc
    base_w = jnp.broadcast_to(
        (starts[None, :] + wbase)[:, :, None], (NW, E, 16)).astype(jnp.int32)

    valid = counts > 0
    ft = jnp.where(valid, starts // T, 0)
    lt = jnp.where(valid, (ends - 1) // T, 0)
    n_e = jnp.where(valid, lt - ft + 1, 0)
    step_start = jnp.concatenate(
        [jnp.zeros((1,), jnp.int32), jnp.cumsum(n_e)[:-1].astype(jnp.int32)])
    nsteps = jnp.sum(n_e)

    g = jnp.arange(G, dtype=jnp.int32)
    in_e = ((g[:, None] >= step_start[None, :])
            & (g[:, None] < (step_start + n_e)[None, :]))
    in_e_i = in_e.astype(jnp.int32)
    e_of_g = jnp.sum(in_e_i * eids[None, :], axis=1)
    start_g = jnp.sum(in_e_i * starts[None, :], axis=1)
    end_g = jnp.sum(in_e_i * ends[None, :], axis=1)
    j = g - jnp.sum(in_e_i * step_start[None, :], axis=1)
    tile = jnp.sum(in_e_i * ft[None, :], axis=1) + j

    is_real = g < nsteps
    tile_ids = jnp.where(is_real, tile, NT - 1).astype(jnp.int32)
    expert_ids = jnp.where(is_real, e_of_g, E - 1).astype(jnp.int32)
    lo = jnp.where(is_real, jnp.maximum(start_g, tile_ids * T), 0).astype(jnp.int32)
    hi = jnp.where(is_real, jnp.minimum(end_g, (tile_ids + 1) * T), 0).astype(jnp.int32)
    first = (is_real & (lo == tile_ids * T)).astype(jnp.int32)
    return base_w, tile_ids, expert_ids, lo, hi, first


def kernel(x, block_indices, W, b):
    xf = x.reshape(N, D)
    idx_w = block_indices.reshape(NW, RPW).astype(jnp.int32)
    base_w, tile_ids, expert_ids, lo, hi, first = _routing_metadata(idx_w)
    scatter_x, gather_out = _sc_kernels()
    xs, pos = scatter_x(xf, idx_w, base_w)
    ys = _grouped_matmul(tile_ids, expert_ids, lo, hi, first,
                         xs, W.astype(jnp.bfloat16).transpose(0, 2, 1),
                         b.reshape(E, 1, P))
    out = gather_out(ys, pos.reshape(NW, NOCH, OCH))
    return out.reshape(B, S, P)
